# initial kernel scaffold (unmeasured)
import functools

import jax
import jax.numpy as jnp
from jax import lax
from jax.experimental import pallas as pl
from jax.experimental.pallas import tpu as pltpu

SERIALIZE = True

NC = 8


def kernel(A, B):
    M, K = A.shape
    K2, N = B.shape
    assert K == K2
    assert N % NC == 0
    CN = N // NC

    A = A.astype(jnp.bfloat16)
    B = B.astype(jnp.bfloat16)

    def body(a_ref, b_ref, out_ref, send_buf, recv_buf, send_sems, recv_sems):
        my_x = lax.axis_index("x")
        my_y = lax.axis_index("y")
        partner = (my_x, 1 - my_y)

        barrier = pltpu.get_barrier_semaphore()
        pl.semaphore_signal(
            barrier, inc=1, device_id=partner,
            device_id_type=pl.DeviceIdType.MESH,
        )
        pl.semaphore_wait(barrier, 1)

        rdmas = []
        for j in range(NC):
            p = jnp.dot(
                a_ref[:, :],
                b_ref[:, j * CN:(j + 1) * CN],
                preferred_element_type=jnp.float32,
            )
            send_buf[j, :, :] = p.astype(jnp.bfloat16)
            rdma = pltpu.make_async_remote_copy(
                src_ref=send_buf.at[j],
                dst_ref=recv_buf.at[j],
                send_sem=send_sems.at[j],
                recv_sem=recv_sems.at[j],
                device_id=partner,
                device_id_type=pl.DeviceIdType.MESH,
            )
            rdma.start()
            if SERIALIZE:
                rdma.wait()
            rdmas.append(rdma)

        for j in range(NC):
            if not SERIALIZE:
                rdmas[j].wait_recv()
            out_ref[:, j * CN:(j + 1) * CN] = (
                send_buf[j, :, :].astype(jnp.float32)
                + recv_buf[j, :, :].astype(jnp.float32)
            )

        if not SERIALIZE:
            for j in range(NC):
                rdmas[j].wait_send()

    return pl.pallas_call(
        body,
        out_shape=jax.ShapeDtypeStruct((M, N), jnp.float32),
        in_specs=[
            pl.BlockSpec(memory_space=pltpu.VMEM),
            pl.BlockSpec(memory_space=pltpu.VMEM),
        ],
        out_specs=pl.BlockSpec(memory_space=pltpu.VMEM),
        scratch_shapes=[
            pltpu.VMEM((NC, M, CN), jnp.bfloat16),
            pltpu.VMEM((NC, M, CN), jnp.bfloat16),
            pltpu.SemaphoreType.DMA((NC,)),
            pltpu.SemaphoreType.DMA((NC,)),
        ],
        compiler_params=pltpu.CompilerParams(collective_id=0),
    )(A, B)


# baseline (device time: 309815 ns/iter reference)
import jax
import jax.numpy as jnp
from jax import lax
from jax.experimental import pallas as pl
from jax.experimental.pallas import tpu as pltpu

SERIALIZE = True

NC = 8


def kernel(A, B):
    M, K = A.shape
    K2, N = B.shape
    assert K == K2
    assert N % NC == 0
    CN = N // NC

    A = A.astype(jnp.bfloat16)
    B = B.astype(jnp.bfloat16)

    def body(a_ref, b_ref, out_ref, recv_buf, send_sems, recv_sems):
        my_x = lax.axis_index("x")
        my_y = lax.axis_index("y")
        partner = (my_x, 1 - my_y)

        barrier = pltpu.get_barrier_semaphore()
        pl.semaphore_signal(
            barrier, inc=1, device_id=partner,
            device_id_type=pl.DeviceIdType.MESH,
        )
        pl.semaphore_wait(barrier, 1)

        rdmas = []
        for j in range(NC):
            cols = pl.ds(j * CN, CN)
            p = jnp.dot(
                a_ref[:, :],
                b_ref[:, cols],
                preferred_element_type=jnp.float32,
            )
            out_ref[:, cols] = p.astype(jnp.bfloat16)
            rdma = pltpu.make_async_remote_copy(
                src_ref=out_ref.at[:, cols],
                dst_ref=recv_buf.at[j],
                send_sem=send_sems.at[j],
                recv_sem=recv_sems.at[j],
                device_id=partner,
                device_id_type=pl.DeviceIdType.MESH,
            )
            rdma.start()
            if SERIALIZE:
                rdma.wait()
            rdmas.append(rdma)

        for j in range(NC):
            cols = pl.ds(j * CN, CN)
            if not SERIALIZE:
                rdmas[j].wait_send()
                rdmas[j].wait_recv()
            out_ref[:, cols] = (
                out_ref[:, cols].astype(jnp.float32)
                + recv_buf[j, :, :].astype(jnp.float32)
            ).astype(jnp.bfloat16)

    return pl.pallas_call(
        body,
        out_shape=jax.ShapeDtypeStruct((M, N), jnp.bfloat16),
        in_specs=[
            pl.BlockSpec(memory_space=pltpu.VMEM),
            pl.BlockSpec(memory_space=pltpu.VMEM),
        ],
        out_specs=pl.BlockSpec(memory_space=pltpu.VMEM),
        scratch_shapes=[
            pltpu.VMEM((NC, M, CN), jnp.bfloat16),
            pltpu.SemaphoreType.DMA((NC,)),
            pltpu.SemaphoreType.DMA((NC,)),
        ],
        compiler_params=pltpu.CompilerParams(
            collective_id=0,
            vmem_limit_bytes=62 * 1024 * 1024,
        ),
    )(A, B)


# device time: 257040 ns/iter; 1.2053x vs baseline; 1.2053x over previous
import jax
import jax.numpy as jnp
from jax import lax
from jax.experimental import pallas as pl
from jax.experimental.pallas import tpu as pltpu

SERIALIZE = False

NC = 8


def kernel(A, B):
    M, K = A.shape
    K2, N = B.shape
    assert K == K2
    assert N % NC == 0
    CN = N // NC

    A = A.astype(jnp.bfloat16)
    B = B.astype(jnp.bfloat16)

    def body(a_ref, b_ref, out_ref, recv_buf, send_sems, recv_sems):
        my_x = lax.axis_index("x")
        my_y = lax.axis_index("y")
        partner = (my_x, 1 - my_y)

        barrier = pltpu.get_barrier_semaphore()
        pl.semaphore_signal(
            barrier, inc=1, device_id=partner,
            device_id_type=pl.DeviceIdType.MESH,
        )
        pl.semaphore_wait(barrier, 1)

        rdmas = []
        for j in range(NC):
            cols = pl.ds(j * CN, CN)
            p = jnp.dot(
                a_ref[:, :],
                b_ref[:, cols],
                preferred_element_type=jnp.float32,
            )
            out_ref[:, cols] = p.astype(jnp.bfloat16)
            rdma = pltpu.make_async_remote_copy(
                src_ref=out_ref.at[:, cols],
                dst_ref=recv_buf.at[j],
                send_sem=send_sems.at[j],
                recv_sem=recv_sems.at[j],
                device_id=partner,
                device_id_type=pl.DeviceIdType.MESH,
            )
            rdma.start()
            if SERIALIZE:
                rdma.wait()
            rdmas.append(rdma)

        for j in range(NC):
            cols = pl.ds(j * CN, CN)
            if not SERIALIZE:
                rdmas[j].wait_send()
                rdmas[j].wait_recv()
            out_ref[:, cols] = (
                out_ref[:, cols].astype(jnp.float32)
                + recv_buf[j, :, :].astype(jnp.float32)
            ).astype(jnp.bfloat16)

    return pl.pallas_call(
        body,
        out_shape=jax.ShapeDtypeStruct((M, N), jnp.bfloat16),
        in_specs=[
            pl.BlockSpec(memory_space=pltpu.VMEM),
            pl.BlockSpec(memory_space=pltpu.VMEM),
        ],
        out_specs=pl.BlockSpec(memory_space=pltpu.VMEM),
        scratch_shapes=[
            pltpu.VMEM((NC, M, CN), jnp.bfloat16),
            pltpu.SemaphoreType.DMA((NC,)),
            pltpu.SemaphoreType.DMA((NC,)),
        ],
        compiler_params=pltpu.CompilerParams(
            collective_id=0,
            vmem_limit_bytes=62 * 1024 * 1024,
        ),
    )(A, B)


# device time: 229014 ns/iter; 1.3528x vs baseline; 1.1224x over previous
import jax
import jax.numpy as jnp
from jax import lax
from jax.experimental import pallas as pl
from jax.experimental.pallas import tpu as pltpu

NB = 4
NP1 = 8


def kernel(A, B):
    M, K = A.shape
    K2, N = B.shape
    assert K == K2
    HALF = N // 2
    CB = HALF // NB
    C1 = N // NP1

    A = A.astype(jnp.bfloat16)
    B = B.astype(jnp.bfloat16)

    def body(a_ref, b_ref, out_ref, a_r, b_r,
             a_sems, b_send, b_recv, f_send, f_recv):
        my_x = lax.axis_index("x")
        my_y = lax.axis_index("y")
        py = (my_x, 1 - my_y)
        px = (1 - my_x, my_y)

        barrier = pltpu.get_barrier_semaphore()
        for nbr in (py, px):
            pl.semaphore_signal(
                barrier, inc=1, device_id=nbr,
                device_id_type=pl.DeviceIdType.MESH,
            )
        pl.semaphore_wait(barrier, 2)

        dbase = my_x * HALF
        fbase = (1 - my_x) * HALF

        a_rdma = pltpu.make_async_remote_copy(
            src_ref=a_ref,
            dst_ref=a_r,
            send_sem=a_sems.at[0],
            recv_sem=a_sems.at[1],
            device_id=py,
            device_id_type=pl.DeviceIdType.MESH,
        )
        a_rdma.start()

        b_rdmas = []
        for c in range(NB):
            cols = pl.ds(dbase + c * CB, CB)
            r = pltpu.make_async_remote_copy(
                src_ref=b_ref.at[:, cols],
                dst_ref=b_r.at[:, cols],
                send_sem=b_send.at[c],
                recv_sem=b_recv.at[c],
                device_id=py,
                device_id_type=pl.DeviceIdType.MESH,
            )
            r.start()
            b_rdmas.append(r)

        for j in range(NP1):
            cols = pl.ds(j * C1, C1)
            out_ref[:, cols] = jnp.dot(
                a_ref[:, :], b_ref[:, cols],
                preferred_element_type=jnp.float32,
            ).astype(jnp.bfloat16)

        a_rdma.wait_recv()

        f_rdmas = []
        for c in range(NB):
            cols = pl.ds(dbase + c * CB, CB)
            b_rdmas[c].wait_recv()
            f = pltpu.make_async_remote_copy(
                src_ref=b_r.at[:, cols],
                dst_ref=b_r.at[:, cols],
                send_sem=f_send.at[c],
                recv_sem=f_recv.at[c],
                device_id=px,
                device_id_type=pl.DeviceIdType.MESH,
            )
            f.start()
            f_rdmas.append(f)
            out_ref[:, cols] = (
                out_ref[:, cols].astype(jnp.float32)
                + jnp.dot(a_r[:, :], b_r[:, cols],
                          preferred_element_type=jnp.float32)
            ).astype(jnp.bfloat16)

        for c in range(NB):
            cols = pl.ds(fbase + c * CB, CB)
            rin = pltpu.make_async_remote_copy(
                src_ref=b_r.at[:, cols],
                dst_ref=b_r.at[:, cols],
                send_sem=f_send.at[c],
                recv_sem=f_recv.at[c],
                device_id=px,
                device_id_type=pl.DeviceIdType.MESH,
            )
            rin.wait_recv()
            out_ref[:, cols] = (
                out_ref[:, cols].astype(jnp.float32)
                + jnp.dot(a_r[:, :], b_r[:, cols],
                          preferred_element_type=jnp.float32)
            ).astype(jnp.bfloat16)

        a_rdma.wait_send()
        for c in range(NB):
            b_rdmas[c].wait_send()
            f_rdmas[c].wait_send()

    return pl.pallas_call(
        body,
        out_shape=jax.ShapeDtypeStruct((M, N), jnp.bfloat16),
        in_specs=[
            pl.BlockSpec(memory_space=pltpu.VMEM),
            pl.BlockSpec(memory_space=pltpu.VMEM),
        ],
        out_specs=pl.BlockSpec(memory_space=pltpu.VMEM),
        scratch_shapes=[
            pltpu.VMEM((M, K), jnp.bfloat16),
            pltpu.VMEM((K, N), jnp.bfloat16),
            pltpu.SemaphoreType.DMA((2,)),
            pltpu.SemaphoreType.DMA((NB,)),
            pltpu.SemaphoreType.DMA((NB,)),
            pltpu.SemaphoreType.DMA((NB,)),
            pltpu.SemaphoreType.DMA((NB,)),
        ],
        compiler_params=pltpu.CompilerParams(
            collective_id=0,
            vmem_limit_bytes=62 * 1024 * 1024,
        ),
    )(A, B)


# device time: 224205 ns/iter; 1.3818x vs baseline; 1.0214x over previous
import jax
import jax.numpy as jnp
from jax import lax
from jax.experimental import pallas as pl
from jax.experimental.pallas import tpu as pltpu

NB = 4
NP1 = 8


def kernel(A, B):
    M, K = A.shape
    K2, N = B.shape
    assert K == K2
    HALF = N // 2
    CB = HALF // NB
    C1 = N // NP1

    A = A.astype(jnp.bfloat16)
    B = B.astype(jnp.bfloat16)

    def body(a_ref, b_ref, out_ref, out_vmem, a_r, b_r,
             a_sems, b_send, b_recv, f_send, f_recv, cp_sems):
        my_x = lax.axis_index("x")
        my_y = lax.axis_index("y")
        py = (my_x, 1 - my_y)
        px = (1 - my_x, my_y)

        barrier = pltpu.get_barrier_semaphore()
        for nbr in (py, px):
            pl.semaphore_signal(
                barrier, inc=1, device_id=nbr,
                device_id_type=pl.DeviceIdType.MESH,
            )
        pl.semaphore_wait(barrier, 2)

        dbase = my_x * HALF
        fbase = (1 - my_x) * HALF

        a_rdma = pltpu.make_async_remote_copy(
            src_ref=a_ref,
            dst_ref=a_r,
            send_sem=a_sems.at[0],
            recv_sem=a_sems.at[1],
            device_id=py,
            device_id_type=pl.DeviceIdType.MESH,
        )
        a_rdma.start()

        b_rdmas = []
        for c in range(NB):
            cols = pl.ds(dbase + c * CB, CB)
            r = pltpu.make_async_remote_copy(
                src_ref=b_ref.at[:, cols],
                dst_ref=b_r.at[:, cols],
                send_sem=b_send.at[c],
                recv_sem=b_recv.at[c],
                device_id=py,
                device_id_type=pl.DeviceIdType.MESH,
            )
            r.start()
            b_rdmas.append(r)

        for j in range(NP1):
            cols = pl.ds(j * C1, C1)
            out_vmem[:, cols] = jnp.dot(
                a_ref[:, :], b_ref[:, cols],
                preferred_element_type=jnp.float32,
            ).astype(jnp.bfloat16)

        a_rdma.wait_recv()

        f_rdmas = []
        copies = []
        for c in range(NB):
            cols = pl.ds(dbase + c * CB, CB)
            b_rdmas[c].wait_recv()
            f = pltpu.make_async_remote_copy(
                src_ref=b_r.at[:, cols],
                dst_ref=b_r.at[:, cols],
                send_sem=f_send.at[c],
                recv_sem=f_recv.at[c],
                device_id=px,
                device_id_type=pl.DeviceIdType.MESH,
            )
            f.start()
            f_rdmas.append(f)
            out_vmem[:, cols] = (
                out_vmem[:, cols].astype(jnp.float32)
                + jnp.dot(a_r[:, :], b_r[:, cols],
                          preferred_element_type=jnp.float32)
            ).astype(jnp.bfloat16)
            cp = pltpu.make_async_copy(
                out_vmem.at[:, cols], out_ref.at[:, cols], cp_sems.at[c]
            )
            cp.start()
            copies.append(cp)

        for c in range(NB):
            cols = pl.ds(fbase + c * CB, CB)
            rin = pltpu.make_async_remote_copy(
                src_ref=b_r.at[:, cols],
                dst_ref=b_r.at[:, cols],
                send_sem=f_send.at[c],
                recv_sem=f_recv.at[c],
                device_id=px,
                device_id_type=pl.DeviceIdType.MESH,
            )
            rin.wait_recv()
            out_vmem[:, cols] = (
                out_vmem[:, cols].astype(jnp.float32)
                + jnp.dot(a_r[:, :], b_r[:, cols],
                          preferred_element_type=jnp.float32)
            ).astype(jnp.bfloat16)
            cp = pltpu.make_async_copy(
                out_vmem.at[:, cols], out_ref.at[:, cols], cp_sems.at[NB + c]
            )
            cp.start()
            copies.append(cp)

        a_rdma.wait_send()
        for c in range(NB):
            b_rdmas[c].wait_send()
            f_rdmas[c].wait_send()
        for cp in copies:
            cp.wait()

    return pl.pallas_call(
        body,
        out_shape=jax.ShapeDtypeStruct((M, N), jnp.bfloat16),
        in_specs=[
            pl.BlockSpec(memory_space=pltpu.VMEM),
            pl.BlockSpec(memory_space=pltpu.VMEM),
        ],
        out_specs=pl.BlockSpec(memory_space=pl.ANY),
        scratch_shapes=[
            pltpu.VMEM((M, N), jnp.bfloat16),
            pltpu.VMEM((M, K), jnp.bfloat16),
            pltpu.VMEM((K, N), jnp.bfloat16),
            pltpu.SemaphoreType.DMA((2,)),
            pltpu.SemaphoreType.DMA((NB,)),
            pltpu.SemaphoreType.DMA((NB,)),
            pltpu.SemaphoreType.DMA((NB,)),
            pltpu.SemaphoreType.DMA((NB,)),
            pltpu.SemaphoreType.DMA((2 * NB,)),
        ],
        compiler_params=pltpu.CompilerParams(
            collective_id=0,
            vmem_limit_bytes=62 * 1024 * 1024,
        ),
    )(A, B)


# device time: 220660 ns/iter; 1.4040x vs baseline; 1.0161x over previous
import jax
import jax.numpy as jnp
from jax import lax
from jax.experimental import pallas as pl
from jax.experimental.pallas import tpu as pltpu

NB = 4
NBC = 8
NA = 4


def kernel(A, B):
    M, K = A.shape
    K2, N = B.shape
    assert K == K2
    HALF = N // 2
    CB = HALF // NB
    KC = K // NA

    A = A.astype(jnp.bfloat16)
    B = B.astype(jnp.bfloat16)

    def body(a_hbm, b_hbm, out_ref, a_vmem, b_vmem, out_vmem, a_r, b_r,
             a_in, b_in, a_send, a_recv, b_send, b_recv, f_send, f_recv,
             cp_sems):
        my_x = lax.axis_index("x")
        my_y = lax.axis_index("y")
        py = (my_x, 1 - my_y)
        px = (1 - my_x, my_y)

        barrier = pltpu.get_barrier_semaphore()
        for nbr in (py, px):
            pl.semaphore_signal(
                barrier, inc=1, device_id=nbr,
                device_id_type=pl.DeviceIdType.MESH,
            )
        pl.semaphore_wait(barrier, 2)

        dbase = my_x * HALF

        def bcol(g):
            return pl.ds(lax.rem(dbase + g * CB, N), CB)

        a_copies = []
        for c in range(NA):
            kc = pl.ds(c * KC, KC)
            cp = pltpu.make_async_copy(
                a_hbm.at[:, kc], a_vmem.at[:, kc], a_in.at[c])
            cp.start()
            a_copies.append(cp)
        b_copies = []
        for g in range(NBC):
            cp = pltpu.make_async_copy(
                b_hbm.at[:, bcol(g)], b_vmem.at[:, bcol(g)], b_in.at[g])
            cp.start()
            b_copies.append(cp)

        a_rdmas = []
        for c in range(NA):
            kc = pl.ds(c * KC, KC)
            a_copies[c].wait()
            r = pltpu.make_async_remote_copy(
                src_ref=a_vmem.at[:, kc],
                dst_ref=a_r.at[:, kc],
                send_sem=a_send.at[c],
                recv_sem=a_recv.at[c],
                device_id=py,
                device_id_type=pl.DeviceIdType.MESH,
            )
            r.start()
            a_rdmas.append(r)

        b_rdmas = []
        for g in range(NB):
            b_copies[g].wait()
            r = pltpu.make_async_remote_copy(
                src_ref=b_vmem.at[:, bcol(g)],
                dst_ref=b_r.at[:, bcol(g)],
                send_sem=b_send.at[g],
                recv_sem=b_recv.at[g],
                device_id=py,
                device_id_type=pl.DeviceIdType.MESH,
            )
            r.start()
            b_rdmas.append(r)

        for g in range(NBC):
            if g >= NB:
                b_copies[g].wait()
            out_vmem[:, bcol(g)] = jnp.dot(
                a_vmem[:, :], b_vmem[:, bcol(g)],
                preferred_element_type=jnp.float32,
            ).astype(jnp.bfloat16)

        for c in range(NA):
            a_rdmas[c].wait_recv()

        f_rdmas = []
        copies = []
        for g in range(NB):
            cols = bcol(g)
            b_rdmas[g].wait_recv()
            f = pltpu.make_async_remote_copy(
                src_ref=b_r.at[:, cols],
                dst_ref=b_r.at[:, cols],
                send_sem=f_send.at[g],
                recv_sem=f_recv.at[g],
                device_id=px,
                device_id_type=pl.DeviceIdType.MESH,
            )
            f.start()
            f_rdmas.append(f)
            out_vmem[:, cols] = (
                out_vmem[:, cols].astype(jnp.float32)
                + jnp.dot(a_r[:, :], b_r[:, cols],
                          preferred_element_type=jnp.float32)
            ).astype(jnp.bfloat16)
            cp = pltpu.make_async_copy(
                out_vmem.at[:, cols], out_ref.at[:, cols], cp_sems.at[g])
            cp.start()
            copies.append(cp)

        for g in range(NB):
            cols = bcol(NB + g)
            rin = pltpu.make_async_remote_copy(
                src_ref=b_r.at[:, cols],
                dst_ref=b_r.at[:, cols],
                send_sem=f_send.at[g],
                recv_sem=f_recv.at[g],
                device_id=px,
                device_id_type=pl.DeviceIdType.MESH,
            )
            rin.wait_recv()
            out_vmem[:, cols] = (
                out_vmem[:, cols].astype(jnp.float32)
                + jnp.dot(a_r[:, :], b_r[:, cols],
                          preferred_element_type=jnp.float32)
            ).astype(jnp.bfloat16)
            cp = pltpu.make_async_copy(
                out_vmem.at[:, cols], out_ref.at[:, cols],
                cp_sems.at[NB + g])
            cp.start()
            copies.append(cp)

        for c in range(NA):
            a_rdmas[c].wait_send()
        for g in range(NB):
            b_rdmas[g].wait_send()
            f_rdmas[g].wait_send()
        for cp in copies:
            cp.wait()

    return pl.pallas_call(
        body,
        out_shape=jax.ShapeDtypeStruct((M, N), jnp.bfloat16),
        in_specs=[
            pl.BlockSpec(memory_space=pl.ANY),
            pl.BlockSpec(memory_space=pl.ANY),
        ],
        out_specs=pl.BlockSpec(memory_space=pl.ANY),
        scratch_shapes=[
            pltpu.VMEM((M, K), jnp.bfloat16),
            pltpu.VMEM((K, N), jnp.bfloat16),
            pltpu.VMEM((M, N), jnp.bfloat16),
            pltpu.VMEM((M, K), jnp.bfloat16),
            pltpu.VMEM((K, N), jnp.bfloat16),
            pltpu.SemaphoreType.DMA((NA,)),
            pltpu.SemaphoreType.DMA((NBC,)),
            pltpu.SemaphoreType.DMA((NA,)),
            pltpu.SemaphoreType.DMA((NA,)),
            pltpu.SemaphoreType.DMA((NB,)),
            pltpu.SemaphoreType.DMA((NB,)),
            pltpu.SemaphoreType.DMA((NB,)),
            pltpu.SemaphoreType.DMA((NB,)),
            pltpu.SemaphoreType.DMA((2 * NB,)),
        ],
        compiler_params=pltpu.CompilerParams(
            collective_id=0,
            vmem_limit_bytes=62 * 1024 * 1024,
        ),
    )(A, B)


# device time: 212182 ns/iter; 1.4601x vs baseline; 1.0400x over previous
import jax
import jax.numpy as jnp
from jax import lax
from jax.experimental import pallas as pl
from jax.experimental.pallas import tpu as pltpu

NB = 4
NBC = 8
NA = 4


def kernel(A, B):
    M, K = A.shape
    K2, N = B.shape
    assert K == K2
    HALF = N // 2
    CB = HALF // NB
    KC = K // NA

    B = B.astype(jnp.bfloat16)

    def body(a_hbm, b_hbm, out_ref, a_vmem, b_vmem, out_vmem, a_r, b_rd,
             a_stage, a_in, b_in, a_send, a_recv, b_send, b_recv,
             f_send, f_recv, cp_sems, ready_sem):
        my_x = lax.axis_index("x")
        my_y = lax.axis_index("y")
        py = (my_x, 1 - my_y)
        px = (1 - my_x, my_y)

        barrier = pltpu.get_barrier_semaphore()
        for nbr in (py, px):
            pl.semaphore_signal(
                barrier, inc=1, device_id=nbr,
                device_id_type=pl.DeviceIdType.MESH,
            )
        pl.semaphore_wait(barrier, 2)

        dbase = my_x * HALF

        def bcol(g):
            return pl.ds(lax.rem(dbase + g * CB, N), CB)

        a_rdmas = []
        for c in range(NA):
            kc = pl.ds(c * KC, KC)
            cp = pltpu.make_async_copy(a_hbm.at[:, kc], a_stage, a_in.at[0])
            cp.start()
            cp.wait()
            a_vmem[:, kc] = a_stage[:, :].astype(jnp.bfloat16)
            r = pltpu.make_async_remote_copy(
                src_ref=a_vmem.at[:, kc],
                dst_ref=a_r.at[:, kc],
                send_sem=a_send.at[c],
                recv_sem=a_recv.at[c],
                device_id=py,
                device_id_type=pl.DeviceIdType.MESH,
            )
            r.start()
            a_rdmas.append(r)

        b_copies = []
        for g in range(NBC):
            cp = pltpu.make_async_copy(
                b_hbm.at[:, bcol(g)], b_vmem.at[:, bcol(g)], b_in.at[g])
            cp.start()
            b_copies.append(cp)
        b_rdmas = []
        for g in range(NBC):
            b_copies[g].wait()
            if g < NB:
                r = pltpu.make_async_remote_copy(
                    src_ref=b_vmem.at[:, bcol(g)],
                    dst_ref=b_rd.at[:, pl.ds(g * CB, CB)],
                    send_sem=b_send.at[g],
                    recv_sem=b_recv.at[g],
                    device_id=py,
                    device_id_type=pl.DeviceIdType.MESH,
                )
                r.start()
                b_rdmas.append(r)

        for g in range(NBC):
            out_vmem[:, bcol(g)] = jnp.dot(
                a_vmem[:, :], b_vmem[:, bcol(g)],
                preferred_element_type=jnp.float32,
            ).astype(jnp.bfloat16)

        pl.semaphore_signal(
            ready_sem, inc=1, device_id=px,
            device_id_type=pl.DeviceIdType.MESH,
        )

        for c in range(NA):
            a_rdmas[c].wait_recv()

        pl.semaphore_wait(ready_sem, 1)

        f_rdmas = []
        copies = []
        for g in range(NB):
            cols = bcol(g)
            lcols = pl.ds(g * CB, CB)
            b_rdmas[g].wait_recv()
            f = pltpu.make_async_remote_copy(
                src_ref=b_rd.at[:, lcols],
                dst_ref=b_vmem.at[:, bcol(g)],
                send_sem=f_send.at[g],
                recv_sem=f_recv.at[g],
                device_id=px,
                device_id_type=pl.DeviceIdType.MESH,
            )
            f.start()
            f_rdmas.append(f)
            out_vmem[:, cols] = (
                out_vmem[:, cols].astype(jnp.float32)
                + jnp.dot(a_r[:, :], b_rd[:, lcols],
                          preferred_element_type=jnp.float32)
            ).astype(jnp.bfloat16)
            cp = pltpu.make_async_copy(
                out_vmem.at[:, cols], out_ref.at[:, cols], cp_sems.at[g])
            cp.start()
            copies.append(cp)

        for g in range(NB):
            cols = bcol(NB + g)
            rin = pltpu.make_async_remote_copy(
                src_ref=b_vmem.at[:, cols],
                dst_ref=b_vmem.at[:, cols],
                send_sem=f_send.at[g],
                recv_sem=f_recv.at[g],
                device_id=px,
                device_id_type=pl.DeviceIdType.MESH,
            )
            rin.wait_recv()
            out_vmem[:, cols] = (
                out_vmem[:, cols].astype(jnp.float32)
                + jnp.dot(a_r[:, :], b_vmem[:, cols],
                          preferred_element_type=jnp.float32)
            ).astype(jnp.bfloat16)
            cp = pltpu.make_async_copy(
                out_vmem.at[:, cols], out_ref.at[:, cols],
                cp_sems.at[NB + g])
            cp.start()
            copies.append(cp)

        for c in range(NA):
            a_rdmas[c].wait_send()
        for g in range(NB):
            b_rdmas[g].wait_send()
            f_rdmas[g].wait_send()
        for cp in copies:
            cp.wait()

    return pl.pallas_call(
        body,
        out_shape=jax.ShapeDtypeStruct((M, N), jnp.bfloat16),
        in_specs=[
            pl.BlockSpec(memory_space=pl.ANY),
            pl.BlockSpec(memory_space=pl.ANY),
        ],
        out_specs=pl.BlockSpec(memory_space=pl.ANY),
        scratch_shapes=[
            pltpu.VMEM((M, K), jnp.bfloat16),
            pltpu.VMEM((K, N), jnp.bfloat16),
            pltpu.VMEM((M, N), jnp.bfloat16),
            pltpu.VMEM((M, K), jnp.bfloat16),
            pltpu.VMEM((K, HALF), jnp.bfloat16),
            pltpu.VMEM((M, KC), jnp.float32),
            pltpu.SemaphoreType.DMA((1,)),
            pltpu.SemaphoreType.DMA((NBC,)),
            pltpu.SemaphoreType.DMA((NA,)),
            pltpu.SemaphoreType.DMA((NA,)),
            pltpu.SemaphoreType.DMA((NB,)),
            pltpu.SemaphoreType.DMA((NB,)),
            pltpu.SemaphoreType.DMA((NB,)),
            pltpu.SemaphoreType.DMA((NB,)),
            pltpu.SemaphoreType.DMA((2 * NB,)),
            pltpu.SemaphoreType.REGULAR,
        ],
        compiler_params=pltpu.CompilerParams(
            collective_id=0,
            vmem_limit_bytes=63 * 1024 * 1024,
        ),
    )(A, B)
